# baseline (speedup 1.0000x reference)
import jax
import jax.numpy as jnp
from jax.experimental import pallas as pl
from jax.experimental.pallas import tpu as pltpu

N = 1024
D = 128
CHUNK = 128
NCHUNK = N // CHUNK
NBUF = 4


def _gin_kernel(a_hbm, x_ref, w_ref, b_ref, o_ref, abuf, sem):
    def copy(i, slot):
        return pltpu.make_async_copy(
            a_hbm.at[pl.ds(i * CHUNK, CHUNK), :], abuf.at[slot], sem.at[slot])

    for s in range(NBUF - 1):
        copy(s, s).start()

    def body(i, _):
        slot = jax.lax.rem(i, NBUF)

        @pl.when(i + NBUF - 1 < NCHUNK)
        def _():
            copy(i + NBUF - 1, jax.lax.rem(i + NBUF - 1, NBUF)).start()

        copy(i, slot).wait()
        aggr = jnp.dot(abuf[slot], x_ref[...], preferred_element_type=jnp.float32)
        h = aggr + x_ref[pl.ds(i * CHUNK, CHUNK), :]
        out = jax.lax.dot_general(h, w_ref[...], (((1,), (1,)), ((), ())),
                                  preferred_element_type=jnp.float32)
        o_ref[pl.ds(i * CHUNK, CHUNK), :] = jnp.maximum(out + b_ref[...], 0.0)
        return ()

    jax.lax.fori_loop(0, NCHUNK, body, (), unroll=False)


def kernel(A, X, W, b):
    return pl.pallas_call(
        _gin_kernel,
        in_specs=[
            pl.BlockSpec(memory_space=pl.ANY),
            pl.BlockSpec((N, D), lambda: (0, 0)),
            pl.BlockSpec((D, D), lambda: (0, 0)),
            pl.BlockSpec((1, D), lambda: (0, 0)),
        ],
        out_specs=pl.BlockSpec((N, D), lambda: (0, 0)),
        out_shape=jax.ShapeDtypeStruct((N, D), jnp.float32),
        scratch_shapes=[
            pltpu.VMEM((NBUF, CHUNK, N), jnp.float32),
            pltpu.SemaphoreType.DMA((NBUF,)),
        ],
    )(A, X, W, b.reshape(1, D))


# speedup vs baseline: 1.0261x; 1.0261x over previous
import jax
import jax.numpy as jnp
from jax.experimental import pallas as pl
from jax.experimental.pallas import tpu as pltpu

N = 1024
D = 128
CHUNK = 128
NCHUNK = N // CHUNK


def _gin_kernel(a_hbm, x_ref, w_ref, b_ref, o_ref, abuf, sem):
    def copy(i):
        return pltpu.make_async_copy(
            a_hbm.at[pl.ds(i * CHUNK, CHUNK), :], abuf.at[i], sem.at[i])

    for i in range(NCHUNK):
        copy(i).start()

    for i in range(NCHUNK):
        copy(i).wait()
        aggr = jnp.dot(abuf[i], x_ref[...], preferred_element_type=jnp.float32)
        h = aggr + x_ref[i * CHUNK:(i + 1) * CHUNK, :]
        out = jax.lax.dot_general(h, w_ref[...], (((1,), (1,)), ((), ())),
                                  preferred_element_type=jnp.float32)
        o_ref[i * CHUNK:(i + 1) * CHUNK, :] = jnp.maximum(out + b_ref[...], 0.0)


def kernel(A, X, W, b):
    return pl.pallas_call(
        _gin_kernel,
        in_specs=[
            pl.BlockSpec(memory_space=pl.ANY),
            pl.BlockSpec((N, D), lambda: (0, 0)),
            pl.BlockSpec((D, D), lambda: (0, 0)),
            pl.BlockSpec((1, D), lambda: (0, 0)),
        ],
        out_specs=pl.BlockSpec((N, D), lambda: (0, 0)),
        out_shape=jax.ShapeDtypeStruct((N, D), jnp.float32),
        scratch_shapes=[
            pltpu.VMEM((NCHUNK, CHUNK, N), jnp.float32),
            pltpu.SemaphoreType.DMA((NCHUNK,)),
        ],
    )(A, X, W, b.reshape(1, D))
